# baseline (speedup 1.0000x reference)
import functools

import jax
import jax.numpy as jnp
from jax import lax
from jax.experimental import pallas as pl
from jax.experimental.pallas import tpu as pltpu
from jax.experimental.pallas import tpu_sc as plsc


def _pack_tc(a2d, b2d):
    def body(a_ref, b_ref, o_ref):
        def rs(v):
            return jnp.where(v == 0.0, jnp.zeros_like(v), lax.rsqrt(v))

        def f16bits(v):
            b = lax.bitcast_convert_type(v, jnp.int32)
            h = (b + 0xFFF + ((b >> 13) & 1)) >> 13
            return jnp.where(b == 0, 0, h - (112 << 10))

        ra = f16bits(rs(a_ref[...]))
        rb = f16bits(rs(b_ref[...]))
        o_ref[...] = ra | (rb << 16)

    return pl.pallas_call(
        body,
        out_shape=jax.ShapeDtypeStruct(a2d.shape, jnp.int32),
    )(a2d, b2d)


_NC = 2
_NS = 16
_NW = _NC * _NS

_F16_BIAS = (127 - 15) << 23


def _f16_half_to_f32(h13):
    return plsc.bitcast(jnp.where(h13 == 0, h13, h13 + _F16_BIAS), jnp.float32)


def _sc_gather_mul(packed, edge_index, chunk, nbuf, unroll):
    n_edges = edge_index.shape[1]
    tot_chunks = n_edges // chunk
    n_groups = chunk // 16
    tbl_n = packed.shape[0]
    n_iters = -(-tot_chunks // _NW)
    n_outer = -(-n_iters // nbuf)
    mesh = plsc.VectorSubcoreMesh(core_axis_name="c", subcore_axis_name="s")

    @functools.partial(
        pl.kernel,
        mesh=mesh,
        out_type=jax.ShapeDtypeStruct((n_edges,), jnp.float32),
        scratch_types=[
            pltpu.VMEM((tbl_n,), jnp.int32),
            pltpu.VMEM((nbuf, 2, chunk), jnp.int32),
            pltpu.VMEM((nbuf, chunk), jnp.float32),
        ]
        + [pltpu.SemaphoreType.DMA] * (2 * nbuf),
        compiler_params=pltpu.CompilerParams(needs_layout_passes=False),
    )
    def k(tbl_hbm, ei_hbm, out_hbm, tbl, ei_v, ob, *sems):
        wid = lax.axis_index("s") * _NC + lax.axis_index("c")
        s_in = sems[:nbuf]
        s_out = sems[nbuf:]

        def in_slice(c):
            return ei_hbm.at[:, pl.ds(pl.multiple_of(c * chunk, 128), chunk)]

        def issue_in(c, b):
            pltpu.async_copy(in_slice(c), ei_v.at[b], s_in[b])

        def wait_in(b):
            pltpu.make_async_copy(in_slice(0), ei_v.at[b], s_in[b]).wait()

        def out_slice(c):
            return out_hbm.at[pl.ds(pl.multiple_of(c * chunk, 128), chunk)]

        def issue_out(c, b):
            pltpu.async_copy(ob.at[b], out_slice(c), s_out[b])

        def wait_out(b):
            pltpu.make_async_copy(ob.at[b], out_slice(0), s_out[b]).wait()

        for b in range(nbuf):
            issue_in(wid + b * _NW, b)
        pltpu.sync_copy(tbl_hbm, tbl)

        def outer(t, carry):
            for b in range(nbuf):
                i = t * nbuf + b
                c = wid + i * _NW

                @pl.when(c < tot_chunks)
                def _():
                    wait_in(b)

                    @pl.when(i >= nbuf)
                    def _():
                        wait_out(b)

                    @plsc.parallel_loop(0, n_groups, unroll=unroll)
                    def grp(j):
                        sl = pl.ds(pl.multiple_of(j * 16, 16), 16)
                        gs = plsc.load_gather(tbl, [ei_v[b, 0, sl]])
                        gd = plsc.load_gather(tbl, [ei_v[b, 1, sl]])
                        f_out = _f16_half_to_f32((gs & 0xFFFF) << 13)
                        f_in = _f16_half_to_f32((gd >> 16) << 13)
                        ob[b, sl] = f_out * f_in

                    issue_out(c, b)

                    @pl.when(c + nbuf * _NW < tot_chunks)
                    def _():
                        issue_in(c + nbuf * _NW, b)

            return carry

        lax.fori_loop(0, n_outer, outer, 0)
        for b in range(nbuf):
            wait_out(b)

    return k(packed, edge_index)


def kernel(out_degree, in_degree, edge_index):
    n = out_degree.shape[0]
    pad = (-n) % 128
    a2d = jnp.pad(out_degree, (0, pad)).reshape(-1, 128)
    b2d = jnp.pad(in_degree, (0, pad)).reshape(-1, 128)
    packed = _pack_tc(a2d, b2d).reshape(-1)
    return _sc_gather_mul(packed, edge_index, chunk=5120, nbuf=2, unroll=10)


# speedup vs baseline: 1.0363x; 1.0363x over previous
import functools

import jax
import jax.numpy as jnp
from jax import lax
from jax.experimental import pallas as pl
from jax.experimental.pallas import tpu as pltpu
from jax.experimental.pallas import tpu_sc as plsc


def _pack_tc(a2d, b2d):
    def body(a_ref, b_ref, o_ref):
        def rs(v):
            return jnp.where(v == 0.0, jnp.zeros_like(v), lax.rsqrt(v))

        def bf16bits(v):
            b = lax.bitcast_convert_type(v, jnp.int32)
            return (b + 0x7FFF + ((b >> 16) & 1)) >> 16

        ra = bf16bits(rs(a_ref[...]))
        rb = bf16bits(rs(b_ref[...]))
        o_ref[...] = ra | (rb << 16)

    return pl.pallas_call(
        body,
        out_shape=jax.ShapeDtypeStruct(a2d.shape, jnp.int32),
    )(a2d, b2d)


_NC = 2
_NS = 16
_NW = _NC * _NS

def _sc_gather_mul(packed, edge_index, chunk, nbuf, unroll):
    n_edges = edge_index.shape[1]
    tot_chunks = n_edges // chunk
    n_groups = chunk // 16
    tbl_n = packed.shape[0]
    n_iters = -(-tot_chunks // _NW)
    n_outer = -(-n_iters // nbuf)
    mesh = plsc.VectorSubcoreMesh(core_axis_name="c", subcore_axis_name="s")

    @functools.partial(
        pl.kernel,
        mesh=mesh,
        out_type=jax.ShapeDtypeStruct((n_edges,), jnp.float32),
        scratch_types=[
            pltpu.VMEM((tbl_n,), jnp.int32),
            pltpu.VMEM((nbuf, 2, chunk), jnp.int32),
            pltpu.VMEM((nbuf, chunk), jnp.float32),
        ]
        + [pltpu.SemaphoreType.DMA] * (2 * nbuf),
        compiler_params=pltpu.CompilerParams(needs_layout_passes=False),
    )
    def k(tbl_hbm, ei_hbm, out_hbm, tbl, ei_v, ob, *sems):
        wid = lax.axis_index("s") * _NC + lax.axis_index("c")
        s_in = sems[:nbuf]
        s_out = sems[nbuf:]

        def in_slice(c):
            return ei_hbm.at[:, pl.ds(pl.multiple_of(c * chunk, 128), chunk)]

        def issue_in(c, b):
            pltpu.async_copy(in_slice(c), ei_v.at[b], s_in[b])

        def wait_in(b):
            pltpu.make_async_copy(in_slice(0), ei_v.at[b], s_in[b]).wait()

        def out_slice(c):
            return out_hbm.at[pl.ds(pl.multiple_of(c * chunk, 128), chunk)]

        def issue_out(c, b):
            pltpu.async_copy(ob.at[b], out_slice(c), s_out[b])

        def wait_out(b):
            pltpu.make_async_copy(ob.at[b], out_slice(0), s_out[b]).wait()

        for b in range(nbuf):
            issue_in(wid + b * _NW, b)
        pltpu.sync_copy(tbl_hbm, tbl)

        def outer(t, carry):
            for b in range(nbuf):
                i = t * nbuf + b
                c = wid + i * _NW

                @pl.when(c < tot_chunks)
                def _():
                    wait_in(b)

                    @pl.when(i >= nbuf)
                    def _():
                        wait_out(b)

                    @plsc.parallel_loop(0, n_groups, unroll=unroll)
                    def grp(j):
                        sl = pl.ds(pl.multiple_of(j * 16, 16), 16)
                        gs = plsc.load_gather(tbl, [ei_v[b, 0, sl]])
                        gd = plsc.load_gather(tbl, [ei_v[b, 1, sl]])
                        f_out = plsc.bitcast(gs << 16, jnp.float32)
                        f_in = plsc.bitcast(gd & -0x10000, jnp.float32)
                        ob[b, sl] = f_out * f_in

                    issue_out(c, b)

                    @pl.when(c + nbuf * _NW < tot_chunks)
                    def _():
                        issue_in(c + nbuf * _NW, b)

            return carry

        lax.fori_loop(0, n_outer, outer, 0)
        for b in range(nbuf):
            wait_out(b)

    return k(packed, edge_index)


def kernel(out_degree, in_degree, edge_index):
    n = out_degree.shape[0]
    pad = (-n) % 128
    a2d = jnp.pad(out_degree, (0, pad)).reshape(-1, 128)
    b2d = jnp.pad(in_degree, (0, pad)).reshape(-1, 128)
    packed = _pack_tc(a2d, b2d).reshape(-1)
    return _sc_gather_mul(packed, edge_index, chunk=5120, nbuf=2, unroll=8)


# speedup vs baseline: 1.0381x; 1.0017x over previous
import functools

import jax
import jax.numpy as jnp
from jax import lax
from jax.experimental import pallas as pl
from jax.experimental.pallas import tpu as pltpu
from jax.experimental.pallas import tpu_sc as plsc


def _pack_tc(a2d, b2d):
    def body(a_ref, b_ref, o_ref):
        def rs(v):
            return jnp.where(v == 0.0, jnp.zeros_like(v), lax.rsqrt(v))

        def bf16bits(v):
            b = lax.bitcast_convert_type(v, jnp.int32)
            return (b + 0x7FFF + ((b >> 16) & 1)) >> 16

        ra = bf16bits(rs(a_ref[...]))
        rb = bf16bits(rs(b_ref[...]))
        o_ref[...] = ra | (rb << 16)

    return pl.pallas_call(
        body,
        out_shape=jax.ShapeDtypeStruct(a2d.shape, jnp.int32),
    )(a2d, b2d)


_NC = 2
_NS = 16
_NW = _NC * _NS

def _sc_gather_mul(packed, edge_index, chunk, nbuf, unroll):
    n_edges = edge_index.shape[1]
    tot_chunks = n_edges // chunk
    n_groups = chunk // 16
    tbl_n = packed.shape[0]
    n_iters = -(-tot_chunks // _NW)
    n_outer = -(-n_iters // nbuf)
    mesh = plsc.VectorSubcoreMesh(core_axis_name="c", subcore_axis_name="s")

    @functools.partial(
        pl.kernel,
        mesh=mesh,
        out_type=jax.ShapeDtypeStruct((n_edges,), jnp.float32),
        scratch_types=[
            pltpu.VMEM((tbl_n,), jnp.int32),
            pltpu.VMEM((nbuf, 2, chunk), jnp.int32),
            pltpu.VMEM((nbuf, chunk), jnp.float32),
        ]
        + [pltpu.SemaphoreType.DMA] * (2 * nbuf),
        compiler_params=pltpu.CompilerParams(needs_layout_passes=False),
    )
    def k(tbl_hbm, ei_hbm, out_hbm, tbl, ei_v, ob, *sems):
        wid = lax.axis_index("s") * _NC + lax.axis_index("c")
        s_in = sems[:nbuf]
        s_out = sems[nbuf:]

        def in_slice(c):
            return ei_hbm.at[:, pl.ds(pl.multiple_of(c * chunk, 128), chunk)]

        def issue_in(c, b):
            pltpu.async_copy(in_slice(c), ei_v.at[b], s_in[b])

        def wait_in(b):
            pltpu.make_async_copy(in_slice(0), ei_v.at[b], s_in[b]).wait()

        def out_slice(c):
            return out_hbm.at[pl.ds(pl.multiple_of(c * chunk, 128), chunk)]

        def issue_out(c, b):
            pltpu.async_copy(ob.at[b], out_slice(c), s_out[b])

        def wait_out(b):
            pltpu.make_async_copy(ob.at[b], out_slice(0), s_out[b]).wait()

        for b in range(nbuf):
            issue_in(wid + b * _NW, b)
        pltpu.sync_copy(tbl_hbm, tbl)

        def outer(t, carry):
            for b in range(nbuf):
                i = t * nbuf + b
                c = wid + i * _NW

                @pl.when(c < tot_chunks)
                def _():
                    wait_in(b)

                    @pl.when(i >= nbuf)
                    def _():
                        wait_out(b)

                    @plsc.parallel_loop(0, n_groups, unroll=unroll)
                    def grp(j):
                        sl = pl.ds(pl.multiple_of(j * 16, 16), 16)
                        gs = plsc.load_gather(tbl, [ei_v[b, 0, sl]])
                        gd = plsc.load_gather(tbl, [ei_v[b, 1, sl]])
                        f_out = plsc.bitcast(gs << 16, jnp.float32)
                        f_in = plsc.bitcast(gd & -0x10000, jnp.float32)
                        ob[b, sl] = f_out * f_in

                    issue_out(c, b)

                    @pl.when(c + nbuf * _NW < tot_chunks)
                    def _():
                        issue_in(c + nbuf * _NW, b)

            return carry

        lax.fori_loop(0, n_outer, outer, 0)
        for b in range(nbuf):
            wait_out(b)

    return k(packed, edge_index)


def kernel(out_degree, in_degree, edge_index):
    n = out_degree.shape[0]
    pad = (-n) % 128
    a2d = jnp.pad(out_degree, (0, pad)).reshape(-1, 128)
    b2d = jnp.pad(in_degree, (0, pad)).reshape(-1, 128)
    packed = _pack_tc(a2d, b2d).reshape(-1)
    return _sc_gather_mul(packed, edge_index, chunk=5120, nbuf=2, unroll=16)
